# baseline (speedup 1.0000x reference)
import functools

import jax
import jax.numpy as jnp
from jax import lax
from jax.experimental import pallas as pl
from jax.experimental.pallas import tpu as pltpu
from jax.experimental.pallas import tpu_sc as plsc

NUM_GATES = 65536
DATA_DIM = 262144
NUM_WORDS = DATA_DIM // 32
NUM_WORKERS = 32
GATES_PER_WORKER = NUM_GATES // NUM_WORKERS
LANES = 16
ITERS = GATES_PER_WORKER // LANES


def _gate_body(table_hbm, a_hbm, b_hbm, out_hbm, table_v, a_v, b_v, o_v):
    wid = lax.axis_index("s") * 2 + lax.axis_index("c")
    base = wid * GATES_PER_WORKER
    pltpu.sync_copy(table_hbm, table_v)
    pltpu.sync_copy(a_hbm.at[pl.ds(base, GATES_PER_WORKER)], a_v)
    pltpu.sync_copy(b_hbm.at[pl.ds(base, GATES_PER_WORKER)], b_v)

    def step(off):
        av = a_v[pl.ds(off, LANES)]
        bv = b_v[pl.ds(off, LANES)]
        ta = plsc.load_gather(table_v, [av >> 5])
        tb = plsc.load_gather(table_v, [bv >> 5])
        ra = (ta >> (av & 31)) & 1
        rb = (tb >> (bv & 31)) & 1
        o_v[pl.ds(off, LANES)] = ra & rb

    def body(i, carry):
        step(i * (4 * LANES))
        step(i * (4 * LANES) + LANES)
        step(i * (4 * LANES) + 2 * LANES)
        step(i * (4 * LANES) + 3 * LANES)
        return carry

    lax.fori_loop(0, ITERS // 4, body, 0)
    pltpu.sync_copy(o_v, out_hbm.at[pl.ds(base, GATES_PER_WORKER)])


_gate_kernel = functools.partial(
    pl.kernel,
    out_type=jax.ShapeDtypeStruct((NUM_GATES,), jnp.int32),
    mesh=plsc.VectorSubcoreMesh(core_axis_name="c", subcore_axis_name="s"),
    scratch_types=[
        pltpu.VMEM((NUM_WORDS,), jnp.int32),
        pltpu.VMEM((GATES_PER_WORKER,), jnp.int32),
        pltpu.VMEM((GATES_PER_WORKER,), jnp.int32),
        pltpu.VMEM((GATES_PER_WORKER,), jnp.int32),
    ],
    compiler_params=pltpu.CompilerParams(needs_layout_passes=False),
)(_gate_body)


def kernel(input_values, input_idxs):
    idx = input_idxs.astype(jnp.int32)
    a = idx[:, 0]
    b = idx[:, 1]
    bits = input_values.reshape(NUM_WORDS, 32).astype(jnp.int32)
    table = jnp.sum(bits << jnp.arange(32, dtype=jnp.int32), axis=1,
                    dtype=jnp.int32)
    out = _gate_kernel(table, a, b)
    return out.astype(bool)


# speedup vs baseline: 1.0334x; 1.0334x over previous
import functools

import jax
import jax.numpy as jnp
from jax import lax
from jax.experimental import pallas as pl
from jax.experimental.pallas import tpu as pltpu
from jax.experimental.pallas import tpu_sc as plsc

NUM_GATES = 65536
DATA_DIM = 262144
NUM_WORDS = DATA_DIM // 32
NUM_WORKERS = 32
GATES_PER_WORKER = NUM_GATES // NUM_WORKERS
LANES = 16
ITERS = GATES_PER_WORKER // LANES


def _gate_body(table_hbm, a_hbm, b_hbm, out_hbm, table_v, a_v, b_v, o_v,
               sem):
    wid = lax.axis_index("s") * 2 + lax.axis_index("c")
    base = wid * GATES_PER_WORKER
    cp_t = pltpu.async_copy(table_hbm, table_v, sem)
    cp_a = pltpu.async_copy(a_hbm.at[pl.ds(base, GATES_PER_WORKER)], a_v, sem)
    cp_b = pltpu.async_copy(b_hbm.at[pl.ds(base, GATES_PER_WORKER)], b_v, sem)
    cp_t.wait()
    cp_a.wait()
    cp_b.wait()

    def step(off):
        av = a_v[pl.ds(off, LANES)]
        bv = b_v[pl.ds(off, LANES)]
        ta = plsc.load_gather(table_v, [av >> 5])
        tb = plsc.load_gather(table_v, [bv >> 5])
        ra = (ta >> (av & 31)) & 1
        rb = (tb >> (bv & 31)) & 1
        o_v[pl.ds(off, LANES)] = ra & rb

    def body(i, carry):
        step(i * (4 * LANES))
        step(i * (4 * LANES) + LANES)
        step(i * (4 * LANES) + 2 * LANES)
        step(i * (4 * LANES) + 3 * LANES)
        return carry

    lax.fori_loop(0, ITERS // 4, body, 0)
    pltpu.sync_copy(o_v, out_hbm.at[pl.ds(base, GATES_PER_WORKER)])


_gate_kernel = functools.partial(
    pl.kernel,
    out_type=jax.ShapeDtypeStruct((NUM_GATES,), jnp.int32),
    mesh=plsc.VectorSubcoreMesh(core_axis_name="c", subcore_axis_name="s"),
    scratch_types=[
        pltpu.VMEM((NUM_WORDS,), jnp.int32),
        pltpu.VMEM((GATES_PER_WORKER,), jnp.int32),
        pltpu.VMEM((GATES_PER_WORKER,), jnp.int32),
        pltpu.VMEM((GATES_PER_WORKER,), jnp.int32),
        pltpu.SemaphoreType.DMA,
    ],
    compiler_params=pltpu.CompilerParams(needs_layout_passes=False),
)(_gate_body)


def kernel(input_values, input_idxs):
    idx = input_idxs.astype(jnp.int32)
    a = idx[:, 0]
    b = idx[:, 1]
    bits = input_values.reshape(NUM_WORDS, 32).astype(jnp.int32)
    table = jnp.sum(bits << jnp.arange(32, dtype=jnp.int32), axis=1,
                    dtype=jnp.int32)
    out = _gate_kernel(table, a, b)
    return out.astype(bool)


# speedup vs baseline: 1.0936x; 1.0583x over previous
import functools

import jax
import jax.numpy as jnp
from jax import lax
from jax.experimental import pallas as pl
from jax.experimental.pallas import tpu as pltpu
from jax.experimental.pallas import tpu_sc as plsc

NUM_GATES = 65536
DATA_DIM = 262144
NUM_WORDS = DATA_DIM // 32
NUM_WORKERS = 32
GATES_PER_WORKER = NUM_GATES // NUM_WORKERS
LANES = 16
ITERS = GATES_PER_WORKER // LANES


def _gate_body(table_hbm, a_hbm, b_hbm, out_hbm, table_v, a_v, b_v, o_v,
               sem):
    wid = lax.axis_index("s") * 2 + lax.axis_index("c")
    base = wid * GATES_PER_WORKER
    cp_t = pltpu.async_copy(table_hbm, table_v, sem)
    cp_a = pltpu.async_copy(a_hbm.at[pl.ds(base, GATES_PER_WORKER)], a_v, sem)
    cp_b = pltpu.async_copy(b_hbm.at[pl.ds(base, GATES_PER_WORKER)], b_v, sem)
    cp_t.wait()
    cp_a.wait()
    cp_b.wait()

    def step(off):
        av = a_v[pl.ds(off, LANES)]
        bv = b_v[pl.ds(off, LANES)]
        ta = plsc.load_gather(table_v, [av & (NUM_WORDS - 1)])
        tb = plsc.load_gather(table_v, [bv & (NUM_WORDS - 1)])
        ra = (ta >> (av >> 13)) & 1
        rb = (tb >> (bv >> 13)) & 1
        o_v[pl.ds(off, LANES)] = ra & rb

    def body(i, carry):
        step(i * (4 * LANES))
        step(i * (4 * LANES) + LANES)
        step(i * (4 * LANES) + 2 * LANES)
        step(i * (4 * LANES) + 3 * LANES)
        return carry

    lax.fori_loop(0, ITERS // 4, body, 0)
    pltpu.sync_copy(o_v, out_hbm.at[pl.ds(base, GATES_PER_WORKER)])


_gate_kernel = functools.partial(
    pl.kernel,
    out_type=jax.ShapeDtypeStruct((NUM_GATES,), jnp.int32),
    mesh=plsc.VectorSubcoreMesh(core_axis_name="c", subcore_axis_name="s"),
    scratch_types=[
        pltpu.VMEM((NUM_WORDS,), jnp.int32),
        pltpu.VMEM((GATES_PER_WORKER,), jnp.int32),
        pltpu.VMEM((GATES_PER_WORKER,), jnp.int32),
        pltpu.VMEM((GATES_PER_WORKER,), jnp.int32),
        pltpu.SemaphoreType.DMA,
    ],
    compiler_params=pltpu.CompilerParams(needs_layout_passes=False),
)(_gate_body)


def kernel(input_values, input_idxs):
    idx = input_idxs.astype(jnp.int32)
    a = idx[:, 0]
    b = idx[:, 1]
    bits = input_values.reshape(32, NUM_WORDS).astype(jnp.int32)
    table = jnp.sum(bits << jnp.arange(32, dtype=jnp.int32)[:, None], axis=0,
                    dtype=jnp.int32)
    out = _gate_kernel(table, a, b)
    return out.astype(bool)
